# Initial kernel scaffold; baseline (speedup 1.0000x reference)
import functools
import jax
import jax.numpy as jnp
from jax import lax
from jax.experimental import pallas as pl
from jax.experimental.pallas import tpu as pltpu
from jax.experimental.pallas import tpu_sc as plsc

NF = 320000
BME = 10000
B = 8
NR = 200
D = 128
NREL = B * NR

NS = 16
K = 80
FPS = NF // NS
NCHUNK = FPS // K
ER = BME // NS
ERC = 125
RR = NREL // NS

_mesh = plsc.VectorSubcoreMesh(core_axis_name="c", subcore_axis_name="s")


@functools.partial(
    pl.kernel,
    mesh=_mesh,
    out_type=jax.ShapeDtypeStruct((2 * NF + 2 * BME + NREL, D), jnp.float32),
    scratch_types=[
        pltpu.VMEM_SHARED((BME, D), jnp.float32),
        pltpu.VMEM_SHARED((NREL, D), jnp.float32),
        pltpu.VMEM((K,), jnp.int32),
        pltpu.VMEM((K,), jnp.int32),
        pltpu.VMEM((K,), jnp.int32),
        pltpu.VMEM((K,), jnp.int32),
        pltpu.VMEM((K,), jnp.float32),
        pltpu.VMEM((K, D), jnp.float32),
        pltpu.VMEM((K, D), jnp.float32),
        pltpu.VMEM((ERC, D), jnp.float32),
        pltpu.SemaphoreType.DMA,
    ],
)
def _gnn_sc(heads, rels, tails, ids, w, ent, ff, out,
            accum, rel_accum,
            idx_v, ridx_v, rels_v, ids_v, w_v, ff_v, ent_v, zb, sem):
    cid = lax.axis_index("c")
    sid = lax.axis_index("s")

    def _zrow(j, _):
        for r in range(D // 16):
            zb[j, pl.ds(r * 16, 16)] = jnp.zeros((16,), jnp.float32)
        return 0
    lax.fori_loop(0, ERC, _zrow, 0)
    for t in range(BME // (NS * ERC)):
        pltpu.sync_copy(zb, accum.at[pl.ds(sid * ER + t * ERC, ERC)])
    pltpu.sync_copy(zb.at[pl.ds(0, RR)], rel_accum.at[pl.ds(sid * RR, RR)])
    plsc.subcore_barrier()

    def chunk_body(g, idx_hbm, gather_base, do_rel):
        base = sid * FPS + g * K
        pltpu.sync_copy(w.at[pl.ds(base, K)], w_v)
        pltpu.sync_copy(idx_hbm.at[pl.ds(base, K)], idx_v)
        pltpu.sync_copy(ff.at[pl.ds(base, K)], ff_v)
        gcp = pltpu.async_copy(ent.at[idx_v], ent_v, sem)
        if do_rel:
            pltpu.sync_copy(rels.at[pl.ds(base, K)], rels_v)
            pltpu.sync_copy(ids.at[pl.ds(base, K)], ids_v)

            def _ridx(t, _):
                sl = pl.ds(t * 16, 16)
                ridx_v[sl] = rels_v[sl] + ids_v[sl] * NR
                return 0
            lax.fori_loop(0, K // 16, _ridx, 0)
        gcp.wait()

        def _fact(j, _):
            wb = plsc.load_gather(w_v, [jnp.full((16,), j, jnp.int32)])
            for r in range(D // 16):
                sl = pl.ds(r * 16, 16)
                ff_v[j, sl] = ff_v[j, sl] * wb
                ent_v[j, sl] = ent_v[j, sl] * wb
            return 0
        lax.fori_loop(0, K, _fact, 0)

        pltpu.sync_copy(ent_v, out.at[pl.ds(gather_base + base, K)])
        pltpu.sync_copy(ff_v, accum.at[idx_v], add=True)
        if do_rel:
            pltpu.sync_copy(ff_v, rel_accum.at[ridx_v], add=True)
        return 0

    @pl.when(cid == 0)
    def _():
        lax.fori_loop(0, NCHUNK, lambda g, c: chunk_body(g, tails, NF, False), 0)

    @pl.when(cid == 1)
    def _():
        lax.fori_loop(0, NCHUNK, lambda g, c: chunk_body(g, heads, 0, True), 0)

    plsc.subcore_barrier()

    ent_base = 2 * NF + cid * BME
    for t in range(BME // (NS * ERC)):
        row = sid * ER + t * ERC
        pltpu.sync_copy(accum.at[pl.ds(row, ERC)],
                        out.at[pl.ds(ent_base + row, ERC)])

    @pl.when(cid == 1)
    def _():
        pltpu.sync_copy(rel_accum.at[pl.ds(sid * RR, RR)],
                        out.at[pl.ds(2 * NF + 2 * BME + sid * RR, RR)])


def kernel(batch_heads, batch_rels, batch_tails, batch_ids, fact_ids,
           weight_list, entity_feat, fact_feat):
    del fact_ids
    i32 = jnp.int32
    return _gnn_sc(batch_heads.astype(i32), batch_rels.astype(i32),
                   batch_tails.astype(i32), batch_ids.astype(i32),
                   weight_list, entity_feat, fact_feat)


# speedup vs baseline: 3.0308x; 3.0308x over previous
import functools
import jax
import jax.numpy as jnp
from jax import lax
from jax.experimental import pallas as pl
from jax.experimental.pallas import tpu as pltpu
from jax.experimental.pallas import tpu_sc as plsc

NF = 320000
BME = 10000
B = 8
NR = 200
D = 128
NREL = B * NR

NS = 16
K = 80
FPS = NF // NS
NCHUNK = FPS // K
AC = 40
NEC = BME // AC
NRC = NREL // AC

_mesh = plsc.VectorSubcoreMesh(core_axis_name="c", subcore_axis_name="s")


@functools.partial(
    pl.kernel,
    mesh=_mesh,
    out_type=jax.ShapeDtypeStruct((2 * NF + 2 * BME + NREL, D), jnp.float32),
    scratch_types=[
        pltpu.VMEM_SHARED((BME, D), jnp.float32),
        pltpu.VMEM_SHARED((NREL, D), jnp.float32),
        pltpu.VMEM((K,), jnp.int32),
        pltpu.VMEM((K,), jnp.int32),
        pltpu.VMEM((K,), jnp.int32),
        pltpu.VMEM((K,), jnp.int32),
        pltpu.VMEM((K + 16,), jnp.float32),
        pltpu.VMEM((K, D), jnp.float32),
        pltpu.VMEM((K, D), jnp.float32),
        pltpu.VMEM((AC, D), jnp.float32),
        pltpu.SemaphoreType.DMA,
    ],
)
def _gnn_sc(heads, rels, tails, ids, w, ent, ff, out,
            accum, rel_accum,
            idx_v, ridx_v, rels_v, ids_v, w_v, ff_v, ent_v, zb, sem):
    cid = lax.axis_index("c")
    sid = lax.axis_index("s")

    def _zrow(j, _):
        for r in range(D // 16):
            zb[j, pl.ds(r * 16, 16)] = jnp.zeros((16,), jnp.float32)
        return 0
    lax.fori_loop(0, AC, _zrow, 0)
    for k in range(-(-NEC // NS)):
        i = k * NS + sid

        @pl.when(i < NEC)
        def _():
            pltpu.sync_copy(zb, accum.at[pl.ds(i * AC, AC)])
    for k in range(-(-NRC // NS)):
        i = k * NS + sid

        @pl.when(i < NRC)
        def _():
            pltpu.sync_copy(zb, rel_accum.at[pl.ds(i * AC, AC)])
    plsc.subcore_barrier()

    def chunk_body(g, idx_hbm, gather_base, do_rel):
        base = sid * FPS + g * K
        pltpu.sync_copy(w.at[pl.ds(base, K)], w_v.at[pl.ds(0, K)])
        pltpu.sync_copy(idx_hbm.at[pl.ds(base, K)], idx_v)
        pltpu.sync_copy(ff.at[pl.ds(base, K)], ff_v)
        gcp = pltpu.async_copy(ent.at[idx_v], ent_v, sem)
        if do_rel:
            pltpu.sync_copy(rels.at[pl.ds(base, K)], rels_v)
            pltpu.sync_copy(ids.at[pl.ds(base, K)], ids_v)

            def _ridx(t, _):
                sl = pl.ds(t * 16, 16)
                ridx_v[sl] = rels_v[sl] + ids_v[sl] * NR
                return 0
            lax.fori_loop(0, K // 16, _ridx, 0)
        gcp.wait()

        def _fact(j, _):
            wb = jnp.full((16,), w_v[pl.ds(j, 16)][0], jnp.float32)
            for r in range(D // 16):
                sl = pl.ds(r * 16, 16)
                ff_v[j, sl] = ff_v[j, sl] * wb
                ent_v[j, sl] = ent_v[j, sl] * wb
            return 0
        lax.fori_loop(0, K, _fact, 0)

        pltpu.sync_copy(ent_v, out.at[pl.ds(gather_base + base, K)])
        pltpu.sync_copy(ff_v, accum.at[idx_v], add=True)
        if do_rel:
            pltpu.sync_copy(ff_v, rel_accum.at[ridx_v], add=True)
        return 0

    @pl.when(cid == 0)
    def _():
        lax.fori_loop(0, NCHUNK, lambda g, c: chunk_body(g, tails, NF, False), 0)

    @pl.when(cid == 1)
    def _():
        lax.fori_loop(0, NCHUNK, lambda g, c: chunk_body(g, heads, 0, True), 0)

    plsc.subcore_barrier()

    ent_base = 2 * NF + cid * BME
    for k in range(-(-NEC // NS)):
        i = k * NS + sid

        @pl.when(i < NEC)
        def _():
            pltpu.sync_copy(accum.at[pl.ds(i * AC, AC)],
                            out.at[pl.ds(ent_base + i * AC, AC)])

    @pl.when(cid == 1)
    def _():
        for k in range(-(-NRC // NS)):
            i = k * NS + sid

            @pl.when(i < NRC)
            def _():
                pltpu.sync_copy(rel_accum.at[pl.ds(i * AC, AC)],
                                out.at[pl.ds(2 * NF + 2 * BME + i * AC, AC)])


def kernel(batch_heads, batch_rels, batch_tails, batch_ids, fact_ids,
           weight_list, entity_feat, fact_feat):
    del fact_ids
    i32 = jnp.int32
    return _gnn_sc(batch_heads.astype(i32), batch_rels.astype(i32),
                   batch_tails.astype(i32), batch_ids.astype(i32),
                   weight_list, entity_feat, fact_feat)


# speedup vs baseline: 3.6292x; 1.1974x over previous
import functools
import jax
import jax.numpy as jnp
from jax import lax
from jax.experimental import pallas as pl
from jax.experimental.pallas import tpu as pltpu
from jax.experimental.pallas import tpu_sc as plsc

NF = 320000
BME = 10000
B = 8
NR = 200
D = 128
NREL = B * NR

NS = 16
K = 40
FPS = NF // NS
NCHUNK = FPS // K
AC = 40
NEC = BME // AC
NRC = NREL // AC

_mesh = plsc.VectorSubcoreMesh(core_axis_name="c", subcore_axis_name="s")


@functools.partial(
    pl.kernel,
    mesh=_mesh,
    out_type=jax.ShapeDtypeStruct((2 * NF + 2 * BME + NREL, D), jnp.float32),
    scratch_types=(
        [pltpu.VMEM_SHARED((BME, D), jnp.float32),
         pltpu.VMEM_SHARED((NREL, D), jnp.float32)]
        + 2 * [pltpu.VMEM((K,), jnp.int32)]
        + 2 * [pltpu.VMEM((K,), jnp.int32)]
        + 2 * [pltpu.VMEM((K,), jnp.int32)]
        + 2 * [pltpu.VMEM((K,), jnp.int32)]
        + 2 * [pltpu.VMEM((K + 16,), jnp.float32)]
        + 2 * [pltpu.VMEM((K, D), jnp.float32)]
        + 2 * [pltpu.VMEM((K, D), jnp.float32)]
        + [pltpu.VMEM((AC, D), jnp.float32)]
        + 6 * [pltpu.SemaphoreType.DMA]
    ),
)
def _gnn_sc(heads, rels, tails, ids, w, ent, ff, out,
            accum, rel_accum,
            idx0, idx1, ridx0, ridx1, rl0, rl1, id0, id1,
            w0, w1, f0, f1, e0, e1, zb,
            insem0, insem1, gsem0, gsem1, osem0, osem1):
    idx_v = [idx0, idx1]
    ridx_v = [ridx0, ridx1]
    rels_v = [rl0, rl1]
    ids_v = [id0, id1]
    w_v = [w0, w1]
    ff_v = [f0, f1]
    ent_v = [e0, e1]
    insem = [insem0, insem1]
    gsem = [gsem0, gsem1]
    osem = [osem0, osem1]
    cid = lax.axis_index("c")
    sid = lax.axis_index("s")

    def _zrow(j, _):
        for r in range(D // 16):
            zb[j, pl.ds(r * 16, 16)] = jnp.zeros((16,), jnp.float32)
        return 0
    lax.fori_loop(0, AC, _zrow, 0)
    for k in range(-(-NEC // NS)):
        i = k * NS + sid

        @pl.when(i < NEC)
        def _():
            pltpu.sync_copy(zb, accum.at[pl.ds(i * AC, AC)])
    for k in range(-(-NRC // NS)):
        i = k * NS + sid

        @pl.when(i < NRC)
        def _():
            pltpu.sync_copy(zb, rel_accum.at[pl.ds(i * AC, AC)])
    plsc.subcore_barrier()

    def run(idx_hbm, gather_base, do_rel):
        def in_copies(g, b):
            base = sid * FPS + g * K
            cs = [
                pltpu.make_async_copy(w.at[pl.ds(base, K)],
                                      w_v[b].at[pl.ds(0, K)], insem[b]),
                pltpu.make_async_copy(idx_hbm.at[pl.ds(base, K)],
                                      idx_v[b], insem[b]),
                pltpu.make_async_copy(ff.at[pl.ds(base, K)], ff_v[b],
                                      insem[b]),
            ]
            if do_rel:
                cs += [
                    pltpu.make_async_copy(rels.at[pl.ds(base, K)],
                                          rels_v[b], insem[b]),
                    pltpu.make_async_copy(ids.at[pl.ds(base, K)],
                                          ids_v[b], insem[b]),
                ]
            return cs

        def issue_in(g, b):
            @pl.when(g < NCHUNK)
            def _():
                for c in in_copies(g, b):
                    c.start()

        def wait_out(g, b):
            @pl.when(g >= 0)
            def _():
                base = sid * FPS + g * K
                pltpu.make_async_copy(
                    ent_v[b], out.at[pl.ds(gather_base + base, K)],
                    osem[b]).wait()
                pltpu.make_async_copy(ff_v[b], accum.at[idx_v[b]],
                                      osem[b]).wait()
                if do_rel:
                    pltpu.make_async_copy(ff_v[b], rel_accum.at[ridx_v[b]],
                                          osem[b]).wait()

        def chunk_body_sync(g, b):
            for c in in_copies(g, b):
                c.wait()
            gcp = pltpu.async_copy(ent.at[idx_v[b]], ent_v[b], gsem[b])
            issue_in(g + 1, 1 - b)
            base = sid * FPS + g * K
            if do_rel:
                for off in sorted({t * 16 for t in range(K // 16)} | {K - 16}):
                    sl = pl.ds(off, 16)
                    ridx_v[b][sl] = rels_v[b][sl] + ids_v[b][sl] * NR
            gcp.wait()

            def _fact(j, _):
                wb = jnp.full((16,), w_v[b][pl.ds(j, 16)][0], jnp.float32)
                for r in range(D // 16):
                    sl = pl.ds(r * 16, 16)
                    ff_v[b][j, sl] = ff_v[b][j, sl] * wb
                    ent_v[b][j, sl] = ent_v[b][j, sl] * wb
                return 0
            lax.fori_loop(0, K, _fact, 0)

            pltpu.sync_copy(ent_v[b],
                            out.at[pl.ds(gather_base + base, K)])
            pltpu.sync_copy(ff_v[b], accum.at[idx_v[b]], add=True)
            if do_rel:
                pltpu.sync_copy(ff_v[b], rel_accum.at[ridx_v[b]], add=True)

        def chunk_body(g, b):
            for c in in_copies(g, b):
                c.wait()
            gcp = pltpu.async_copy(ent.at[idx_v[b]], ent_v[b], gsem[b])
            wait_out(g - 1, 1 - b)
            issue_in(g + 1, 1 - b)
            if do_rel:
                for off in sorted({t * 16 for t in range(K // 16)} | {K - 16}):
                    sl = pl.ds(off, 16)
                    ridx_v[b][sl] = rels_v[b][sl] + ids_v[b][sl] * NR
            gcp.wait()

            def _fact(j, _):
                wb = jnp.full((16,), w_v[b][pl.ds(j, 16)][0], jnp.float32)
                for r in range(D // 16):
                    sl = pl.ds(r * 16, 16)
                    ff_v[b][j, sl] = ff_v[b][j, sl] * wb
                    ent_v[b][j, sl] = ent_v[b][j, sl] * wb
                return 0
            lax.fori_loop(0, K, _fact, 0)

            base = sid * FPS + g * K
            pltpu.async_copy(ent_v[b],
                             out.at[pl.ds(gather_base + base, K)], osem[b])
            pltpu.async_copy(ff_v[b], accum.at[idx_v[b]], osem[b], add=True)
            if do_rel:
                pltpu.async_copy(ff_v[b], rel_accum.at[ridx_v[b]],
                                 osem[b], add=True)

        issue_in(0, 0)

        def outer(i, _):
            chunk_body_sync(2 * i, 0)
            chunk_body_sync(2 * i + 1, 1)
            return 0
        lax.fori_loop(0, NCHUNK // 2, outer, 0)

    @pl.when(cid == 0)
    def _():
        run(tails, NF, False)

    @pl.when(cid == 1)
    def _():
        run(heads, 0, True)

    plsc.subcore_barrier()

    ent_base = 2 * NF + cid * BME
    for k in range(-(-NEC // NS)):
        i = k * NS + sid

        @pl.when(i < NEC)
        def _():
            pltpu.sync_copy(accum.at[pl.ds(i * AC, AC)],
                            out.at[pl.ds(ent_base + i * AC, AC)])

    @pl.when(cid == 1)
    def _():
        for k in range(-(-NRC // NS)):
            i = k * NS + sid

            @pl.when(i < NRC)
            def _():
                pltpu.sync_copy(rel_accum.at[pl.ds(i * AC, AC)],
                                out.at[pl.ds(2 * NF + 2 * BME + i * AC, AC)])


def kernel(batch_heads, batch_rels, batch_tails, batch_ids, fact_ids,
           weight_list, entity_feat, fact_feat):
    del fact_ids
    i32 = jnp.int32
    return _gnn_sc(batch_heads.astype(i32), batch_rels.astype(i32),
                   batch_tails.astype(i32), batch_ids.astype(i32),
                   weight_list, entity_feat, fact_feat)


# speedup vs baseline: 3.9190x; 1.0798x over previous
import functools
import jax
import jax.numpy as jnp
from jax import lax
from jax.experimental import pallas as pl
from jax.experimental.pallas import tpu as pltpu
from jax.experimental.pallas import tpu_sc as plsc

NF = 320000
BME = 10000
B = 8
NR = 200
D = 128
NREL = B * NR

NS = 16
K = 40
FPS = NF // NS
NCHUNK = FPS // K
AC = 40
NEC = BME // AC
NRC = NREL // AC

_mesh = plsc.VectorSubcoreMesh(core_axis_name="c", subcore_axis_name="s")


@functools.partial(
    pl.kernel,
    mesh=_mesh,
    out_type=jax.ShapeDtypeStruct((2 * NF + 2 * BME + NREL, D), jnp.float32),
    scratch_types=(
        [pltpu.VMEM_SHARED((BME, D), jnp.float32),
         pltpu.VMEM_SHARED((NREL, D), jnp.float32)]
        + 2 * [pltpu.VMEM((K,), jnp.int32)]
        + 2 * [pltpu.VMEM((K,), jnp.int32)]
        + 2 * [pltpu.VMEM((K,), jnp.int32)]
        + 2 * [pltpu.VMEM((K,), jnp.int32)]
        + 2 * [pltpu.VMEM((K + 16,), jnp.float32)]
        + 2 * [pltpu.VMEM((K, D), jnp.float32)]
        + 2 * [pltpu.VMEM((K, D), jnp.float32)]
        + [pltpu.VMEM((AC, D), jnp.float32)]
        + 6 * [pltpu.SemaphoreType.DMA]
    ),
)
def _gnn_sc(heads, rels, tails, ids, w, ent, ff, out,
            accum, rel_accum,
            idx0, idx1, ridx0, ridx1, rl0, rl1, id0, id1,
            w0, w1, f0, f1, e0, e1, zb,
            insem0, insem1, gsem0, gsem1, osem0, osem1):
    idx_v = [idx0, idx1]
    ridx_v = [ridx0, ridx1]
    rels_v = [rl0, rl1]
    ids_v = [id0, id1]
    w_v = [w0, w1]
    ff_v = [f0, f1]
    ent_v = [e0, e1]
    insem = [insem0, insem1]
    gsem = [gsem0, gsem1]
    osem = [osem0, osem1]
    cid = lax.axis_index("c")
    sid = lax.axis_index("s")

    def _zrow(j, _):
        for r in range(D // 16):
            zb[j, pl.ds(r * 16, 16)] = jnp.zeros((16,), jnp.float32)
        return 0
    lax.fori_loop(0, AC, _zrow, 0)
    for k in range(-(-NEC // NS)):
        i = k * NS + sid

        @pl.when(i < NEC)
        def _():
            pltpu.sync_copy(zb, accum.at[pl.ds(i * AC, AC)])
    for k in range(-(-NRC // NS)):
        i = k * NS + sid

        @pl.when(i < NRC)
        def _():
            pltpu.sync_copy(zb, rel_accum.at[pl.ds(i * AC, AC)])
    plsc.subcore_barrier()

    def run(idx_hbm, gather_base, do_rel):
        def in_copies(g, b):
            base = sid * FPS + g * K
            cs = [
                pltpu.make_async_copy(w.at[pl.ds(base, K)],
                                      w_v[b].at[pl.ds(0, K)], insem[b]),
                pltpu.make_async_copy(idx_hbm.at[pl.ds(base, K)],
                                      idx_v[b], insem[b]),
                pltpu.make_async_copy(ff.at[pl.ds(base, K)], ff_v[b],
                                      insem[b]),
            ]
            if do_rel:
                cs += [
                    pltpu.make_async_copy(rels.at[pl.ds(base, K)],
                                          rels_v[b], insem[b]),
                    pltpu.make_async_copy(ids.at[pl.ds(base, K)],
                                          ids_v[b], insem[b]),
                ]
            return cs

        def issue_in(g, b):
            @pl.when(g < NCHUNK)
            def _():
                for c in in_copies(g, b):
                    c.start()

        def wait_out(g, b):
            @pl.when(g >= 0)
            def _():
                base = sid * FPS + g * K
                pltpu.make_async_copy(
                    ent_v[b], out.at[pl.ds(gather_base + base, K)],
                    osem[b]).wait()
                pltpu.make_async_copy(ff_v[b], accum.at[idx_v[b]],
                                      osem[b]).wait()
                if do_rel:
                    pltpu.make_async_copy(ff_v[b], rel_accum.at[ridx_v[b]],
                                          osem[b]).wait()

        def wait_out_lin(g, b):
            @pl.when(g >= 0)
            def _():
                base = sid * FPS + g * K
                pltpu.make_async_copy(
                    ent_v[b], out.at[pl.ds(gather_base + base, K)],
                    osem[b]).wait()

        def chunk_body_sync(g, b):
            for c in in_copies(g, b):
                c.wait()
            wait_out_lin(g - 2, b)
            gcp = pltpu.async_copy(ent.at[idx_v[b]], ent_v[b], gsem[b])
            issue_in(g + 1, 1 - b)
            base = sid * FPS + g * K
            if do_rel:
                for off in sorted({t * 16 for t in range(K // 16)} | {K - 16}):
                    sl = pl.ds(off, 16)
                    ridx_v[b][sl] = rels_v[b][sl] + ids_v[b][sl] * NR
            gcp.wait()

            def _fact(j, _):
                wb = jnp.full((16,), w_v[b][pl.ds(j, 16)][0], jnp.float32)
                for r in range(D // 16):
                    sl = pl.ds(r * 16, 16)
                    ff_v[b][j, sl] = ff_v[b][j, sl] * wb
                    ent_v[b][j, sl] = ent_v[b][j, sl] * wb
                return 0
            lax.fori_loop(0, K, _fact, 0)

            pltpu.async_copy(ent_v[b],
                             out.at[pl.ds(gather_base + base, K)], osem[b])
            pltpu.sync_copy(ff_v[b], accum.at[idx_v[b]], add=True)
            if do_rel:
                pltpu.sync_copy(ff_v[b], rel_accum.at[ridx_v[b]], add=True)

        def chunk_body(g, b):
            for c in in_copies(g, b):
                c.wait()
            gcp = pltpu.async_copy(ent.at[idx_v[b]], ent_v[b], gsem[b])
            wait_out(g - 1, 1 - b)
            issue_in(g + 1, 1 - b)
            if do_rel:
                for off in sorted({t * 16 for t in range(K // 16)} | {K - 16}):
                    sl = pl.ds(off, 16)
                    ridx_v[b][sl] = rels_v[b][sl] + ids_v[b][sl] * NR
            gcp.wait()

            def _fact(j, _):
                wb = jnp.full((16,), w_v[b][pl.ds(j, 16)][0], jnp.float32)
                for r in range(D // 16):
                    sl = pl.ds(r * 16, 16)
                    ff_v[b][j, sl] = ff_v[b][j, sl] * wb
                    ent_v[b][j, sl] = ent_v[b][j, sl] * wb
                return 0
            lax.fori_loop(0, K, _fact, 0)

            base = sid * FPS + g * K
            pltpu.async_copy(ent_v[b],
                             out.at[pl.ds(gather_base + base, K)], osem[b])
            pltpu.async_copy(ff_v[b], accum.at[idx_v[b]], osem[b], add=True)
            if do_rel:
                pltpu.async_copy(ff_v[b], rel_accum.at[ridx_v[b]],
                                 osem[b], add=True)

        issue_in(0, 0)

        def outer(i, _):
            chunk_body_sync(2 * i, 0)
            chunk_body_sync(2 * i + 1, 1)
            return 0
        lax.fori_loop(0, NCHUNK // 2, outer, 0)
        wait_out_lin(NCHUNK - 2, 0)
        wait_out_lin(NCHUNK - 1, 1)

    @pl.when(cid == 0)
    def _():
        run(tails, NF, False)

    @pl.when(cid == 1)
    def _():
        run(heads, 0, True)

    plsc.subcore_barrier()

    ent_base = 2 * NF + cid * BME
    for k in range(-(-NEC // NS)):
        i = k * NS + sid

        @pl.when(i < NEC)
        def _():
            pltpu.sync_copy(accum.at[pl.ds(i * AC, AC)],
                            out.at[pl.ds(ent_base + i * AC, AC)])

    @pl.when(cid == 1)
    def _():
        for k in range(-(-NRC // NS)):
            i = k * NS + sid

            @pl.when(i < NRC)
            def _():
                pltpu.sync_copy(rel_accum.at[pl.ds(i * AC, AC)],
                                out.at[pl.ds(2 * NF + 2 * BME + i * AC, AC)])


def kernel(batch_heads, batch_rels, batch_tails, batch_ids, fact_ids,
           weight_list, entity_feat, fact_feat):
    del fact_ids
    i32 = jnp.int32
    return _gnn_sc(batch_heads.astype(i32), batch_rels.astype(i32),
                   batch_tails.astype(i32), batch_ids.astype(i32),
                   weight_list, entity_feat, fact_feat)


# speedup vs baseline: 5.1399x; 1.3115x over previous
import functools
import jax
import jax.numpy as jnp
from jax import lax
from jax.experimental import pallas as pl
from jax.experimental.pallas import tpu as pltpu
from jax.experimental.pallas import tpu_sc as plsc

NF = 320000
BME = 10000
B = 8
NR = 200
D = 128
NREL = B * NR

NS = 16
K = 40
FPS = NF // NS
NCHUNK = FPS // K
AC = 40
NEC = BME // AC
NRC = NREL // AC

_mesh = plsc.VectorSubcoreMesh(core_axis_name="c", subcore_axis_name="s")


@functools.partial(
    pl.kernel,
    mesh=_mesh,
    out_type=jax.ShapeDtypeStruct((2 * NF + 2 * BME + NREL, D), jnp.float32),
    scratch_types=(
        [pltpu.VMEM_SHARED((BME, D), jnp.float32),
         pltpu.VMEM_SHARED((NREL, D), jnp.float32)]
        + 2 * [pltpu.VMEM((K,), jnp.int32)]
        + 2 * [pltpu.VMEM((K,), jnp.int32)]
        + 2 * [pltpu.VMEM((K,), jnp.int32)]
        + 2 * [pltpu.VMEM((K,), jnp.int32)]
        + 2 * [pltpu.VMEM((K + 16,), jnp.float32)]
        + 2 * [pltpu.VMEM((K, D), jnp.float32)]
        + 2 * [pltpu.VMEM((K, D), jnp.float32)]
        + [pltpu.VMEM((AC, D), jnp.float32)]
        + 8 * [pltpu.SemaphoreType.DMA]
    ),
)
def _gnn_sc(heads, rels, tails, ids, w, ent, ff, out,
            accum, rel_accum,
            idx0, idx1, ridx0, ridx1, rl0, rl1, id0, id1,
            w0, w1, f0, f1, e0, e1, zb,
            insem0, insem1, gsem0, gsem1, osem0, osem1, asem0, asem1):
    idx_v = [idx0, idx1]
    ridx_v = [ridx0, ridx1]
    rels_v = [rl0, rl1]
    ids_v = [id0, id1]
    w_v = [w0, w1]
    ff_v = [f0, f1]
    ent_v = [e0, e1]
    insem = [insem0, insem1]
    gsem = [gsem0, gsem1]
    osem = [osem0, osem1]
    asem = [asem0, asem1]
    cid = lax.axis_index("c")
    sid = lax.axis_index("s")

    def _zrow(j, _):
        for r in range(D // 16):
            zb[j, pl.ds(r * 16, 16)] = jnp.zeros((16,), jnp.float32)
        return 0
    lax.fori_loop(0, AC, _zrow, 0)
    for k in range(-(-NEC // NS)):
        i = k * NS + sid

        @pl.when(i < NEC)
        def _():
            pltpu.sync_copy(zb, accum.at[pl.ds(i * AC, AC)])
    for k in range(-(-NRC // NS)):
        i = k * NS + sid

        @pl.when(i < NRC)
        def _():
            pltpu.sync_copy(zb, rel_accum.at[pl.ds(i * AC, AC)])
    plsc.subcore_barrier()

    def run(idx_hbm, gather_base, do_rel):
        def in_copies(g, b):
            base = sid * FPS + g * K
            cs = [
                pltpu.make_async_copy(w.at[pl.ds(base, K)],
                                      w_v[b].at[pl.ds(0, K)], insem[b]),
                pltpu.make_async_copy(idx_hbm.at[pl.ds(base, K)],
                                      idx_v[b], insem[b]),
                pltpu.make_async_copy(ff.at[pl.ds(base, K)], ff_v[b],
                                      insem[b]),
            ]
            if do_rel:
                cs += [
                    pltpu.make_async_copy(rels.at[pl.ds(base, K)],
                                          rels_v[b], insem[b]),
                    pltpu.make_async_copy(ids.at[pl.ds(base, K)],
                                          ids_v[b], insem[b]),
                ]
            return cs

        def issue_in(g, b):
            @pl.when(g < NCHUNK)
            def _():
                for c in in_copies(g, b):
                    c.start()

        def wait_out(g, b):
            @pl.when(g >= 0)
            def _():
                base = sid * FPS + g * K
                pltpu.make_async_copy(
                    ent_v[b], out.at[pl.ds(gather_base + base, K)],
                    osem[b]).wait()
                pltpu.make_async_copy(ff_v[b], accum.at[idx_v[b]],
                                      osem[b]).wait()
                if do_rel:
                    pltpu.make_async_copy(ff_v[b], rel_accum.at[ridx_v[b]],
                                          osem[b]).wait()

        def wait_out_lin(g, b):
            @pl.when(g >= 0)
            def _():
                base = sid * FPS + g * K
                pltpu.make_async_copy(
                    ent_v[b], out.at[pl.ds(gather_base + base, K)],
                    osem[b]).wait()

        def wait_adds(g, b):
            @pl.when(g >= 0)
            def _():
                pltpu.make_async_copy(ff_v[b], accum.at[idx_v[b]],
                                      asem[b]).wait()
                if do_rel:
                    pltpu.make_async_copy(ff_v[b], rel_accum.at[ridx_v[b]],
                                          asem[b]).wait()

        def chunk_body_sync(g, b):
            for c in in_copies(g, b):
                c.wait()
            wait_out_lin(g - 2, b)
            gcp = pltpu.async_copy(ent.at[idx_v[b]], ent_v[b], gsem[b])
            wait_adds(g - 1, 1 - b)
            issue_in(g + 1, 1 - b)
            base = sid * FPS + g * K
            if do_rel:
                for off in sorted({t * 16 for t in range(K // 16)} | {K - 16}):
                    sl = pl.ds(off, 16)
                    ridx_v[b][sl] = rels_v[b][sl] + ids_v[b][sl] * NR
            gcp.wait()

            def _fact(j, _):
                wb = jnp.full((16,), w_v[b][pl.ds(j, 16)][0], jnp.float32)
                for r in range(D // 16):
                    sl = pl.ds(r * 16, 16)
                    ff_v[b][j, sl] = ff_v[b][j, sl] * wb
                    ent_v[b][j, sl] = ent_v[b][j, sl] * wb
                return 0
            lax.fori_loop(0, K, _fact, 0)

            pltpu.async_copy(ent_v[b],
                             out.at[pl.ds(gather_base + base, K)], osem[b])
            pltpu.async_copy(ff_v[b], accum.at[idx_v[b]], asem[b], add=True)
            if do_rel:
                pltpu.async_copy(ff_v[b], rel_accum.at[ridx_v[b]],
                                 asem[b], add=True)

        def chunk_body(g, b):
            for c in in_copies(g, b):
                c.wait()
            gcp = pltpu.async_copy(ent.at[idx_v[b]], ent_v[b], gsem[b])
            wait_out(g - 1, 1 - b)
            issue_in(g + 1, 1 - b)
            if do_rel:
                for off in sorted({t * 16 for t in range(K // 16)} | {K - 16}):
                    sl = pl.ds(off, 16)
                    ridx_v[b][sl] = rels_v[b][sl] + ids_v[b][sl] * NR
            gcp.wait()

            def _fact(j, _):
                wb = jnp.full((16,), w_v[b][pl.ds(j, 16)][0], jnp.float32)
                for r in range(D // 16):
                    sl = pl.ds(r * 16, 16)
                    ff_v[b][j, sl] = ff_v[b][j, sl] * wb
                    ent_v[b][j, sl] = ent_v[b][j, sl] * wb
                return 0
            lax.fori_loop(0, K, _fact, 0)

            base = sid * FPS + g * K
            pltpu.async_copy(ent_v[b],
                             out.at[pl.ds(gather_base + base, K)], osem[b])
            pltpu.async_copy(ff_v[b], accum.at[idx_v[b]], osem[b], add=True)
            if do_rel:
                pltpu.async_copy(ff_v[b], rel_accum.at[ridx_v[b]],
                                 osem[b], add=True)

        issue_in(0, 0)

        def outer(i, _):
            chunk_body_sync(2 * i, 0)
            chunk_body_sync(2 * i + 1, 1)
            return 0
        lax.fori_loop(0, NCHUNK // 2, outer, 0)
        wait_out_lin(NCHUNK - 2, 0)
        wait_out_lin(NCHUNK - 1, 1)
        wait_adds(NCHUNK - 1, 1)

    @pl.when(cid == 0)
    def _():
        run(tails, NF, False)

    @pl.when(cid == 1)
    def _():
        run(heads, 0, True)

    plsc.subcore_barrier()

    ent_base = 2 * NF + cid * BME
    for k in range(-(-NEC // NS)):
        i = k * NS + sid

        @pl.when(i < NEC)
        def _():
            pltpu.sync_copy(accum.at[pl.ds(i * AC, AC)],
                            out.at[pl.ds(ent_base + i * AC, AC)])

    @pl.when(cid == 1)
    def _():
        for k in range(-(-NRC // NS)):
            i = k * NS + sid

            @pl.when(i < NRC)
            def _():
                pltpu.sync_copy(rel_accum.at[pl.ds(i * AC, AC)],
                                out.at[pl.ds(2 * NF + 2 * BME + i * AC, AC)])


def kernel(batch_heads, batch_rels, batch_tails, batch_ids, fact_ids,
           weight_list, entity_feat, fact_feat):
    del fact_ids
    i32 = jnp.int32
    return _gnn_sc(batch_heads.astype(i32), batch_rels.astype(i32),
                   batch_tails.astype(i32), batch_ids.astype(i32),
                   weight_list, entity_feat, fact_feat)


# speedup vs baseline: 5.1432x; 1.0006x over previous
import functools
import jax
import jax.numpy as jnp
from jax import lax
from jax.experimental import pallas as pl
from jax.experimental.pallas import tpu as pltpu
from jax.experimental.pallas import tpu_sc as plsc

NF = 320000
BME = 10000
B = 8
NR = 200
D = 128
NREL = B * NR

NS = 16
K = 40
FPS = NF // NS
NCHUNK = FPS // K
AC = 40
NEC = BME // AC
NRC = NREL // AC

_mesh = plsc.VectorSubcoreMesh(core_axis_name="c", subcore_axis_name="s")


@functools.partial(
    pl.kernel,
    mesh=_mesh,
    out_type=jax.ShapeDtypeStruct((2 * NF + 2 * BME + NREL, D), jnp.float32),
    scratch_types=(
        [pltpu.VMEM_SHARED((BME, D), jnp.float32),
         pltpu.VMEM_SHARED((NREL, D), jnp.float32)]
        + 2 * [pltpu.VMEM((K,), jnp.int32)]
        + 2 * [pltpu.VMEM((K,), jnp.int32)]
        + 2 * [pltpu.VMEM((K,), jnp.int32)]
        + 2 * [pltpu.VMEM((K,), jnp.int32)]
        + 2 * [pltpu.VMEM((K + 16,), jnp.float32)]
        + 2 * [pltpu.VMEM((K, D), jnp.float32)]
        + 2 * [pltpu.VMEM((K, D), jnp.float32)]
        + [pltpu.VMEM((AC, D), jnp.float32)]
        + 8 * [pltpu.SemaphoreType.DMA]
    ),
)
def _gnn_sc(heads, rels, tails, ids, w, ent, ff, out,
            accum, rel_accum,
            idx0, idx1, ridx0, ridx1, rl0, rl1, id0, id1,
            w0, w1, f0, f1, e0, e1, zb,
            insem0, insem1, gsem0, gsem1, osem0, osem1, asem0, asem1):
    idx_v = [idx0, idx1]
    ridx_v = [ridx0, ridx1]
    rels_v = [rl0, rl1]
    ids_v = [id0, id1]
    w_v = [w0, w1]
    ff_v = [f0, f1]
    ent_v = [e0, e1]
    insem = [insem0, insem1]
    gsem = [gsem0, gsem1]
    osem = [osem0, osem1]
    asem = [asem0, asem1]
    cid = lax.axis_index("c")
    sid = lax.axis_index("s")

    def _zrow(j, _):
        for r in range(D // 16):
            zb[j, pl.ds(r * 16, 16)] = jnp.zeros((16,), jnp.float32)
        return 0
    lax.fori_loop(0, AC, _zrow, 0)
    for k in range(-(-NEC // NS)):
        i = k * NS + sid

        @pl.when(i < NEC)
        def _():
            pltpu.sync_copy(zb, accum.at[pl.ds(i * AC, AC)])
    for k in range(-(-NRC // NS)):
        i = k * NS + sid

        @pl.when(i < NRC)
        def _():
            pltpu.sync_copy(zb, rel_accum.at[pl.ds(i * AC, AC)])
    plsc.subcore_barrier()

    def run(idx_hbm, gather_base, do_rel):
        def in_copies(g, b):
            base = sid * FPS + g * K
            cs = [
                pltpu.make_async_copy(w.at[pl.ds(base, K)],
                                      w_v[b].at[pl.ds(0, K)], insem[b]),
                pltpu.make_async_copy(idx_hbm.at[pl.ds(base, K)],
                                      idx_v[b], insem[b]),
                pltpu.make_async_copy(ff.at[pl.ds(base, K)], ff_v[b],
                                      insem[b]),
            ]
            if do_rel:
                cs += [
                    pltpu.make_async_copy(rels.at[pl.ds(base, K)],
                                          rels_v[b], insem[b]),
                    pltpu.make_async_copy(ids.at[pl.ds(base, K)],
                                          ids_v[b], insem[b]),
                ]
            return cs

        def issue_in(g, b):
            @pl.when(g < NCHUNK)
            def _():
                for c in in_copies(g, b):
                    c.start()

        def wait_out(g, b):
            @pl.when(g >= 0)
            def _():
                base = sid * FPS + g * K
                pltpu.make_async_copy(
                    ent_v[b], out.at[pl.ds(gather_base + base, K)],
                    osem[b]).wait()
                pltpu.make_async_copy(ff_v[b], accum.at[idx_v[b]],
                                      osem[b]).wait()
                if do_rel:
                    pltpu.make_async_copy(ff_v[b], rel_accum.at[ridx_v[b]],
                                          osem[b]).wait()

        def wait_out_lin(g, b):
            @pl.when(g >= 0)
            def _():
                base = sid * FPS + g * K
                pltpu.make_async_copy(
                    ent_v[b], out.at[pl.ds(gather_base + base, K)],
                    osem[b]).wait()

        def wait_adds(g, b):
            @pl.when(g >= 0)
            def _():
                pltpu.make_async_copy(ff_v[b], accum.at[idx_v[b]],
                                      asem[b]).wait()
                if do_rel:
                    pltpu.make_async_copy(ff_v[b], rel_accum.at[ridx_v[b]],
                                          asem[b]).wait()

        def chunk_body(g, b):
            wait_adds(g - 1, 1 - b)
            issue_in(g + 1, 1 - b)
            base = sid * FPS + g * K
            if do_rel:
                for off in sorted({t * 16 for t in range(K // 16)} | {K - 16}):
                    sl = pl.ds(off, 16)
                    ridx_v[b][sl] = rels_v[b][sl] + ids_v[b][sl] * NR
            pltpu.make_async_copy(ent.at[idx_v[b]], ent_v[b], gsem[b]).wait()

            def _fact(j, _):
                wb = jnp.full((16,), w_v[b][pl.ds(j, 16)][0], jnp.float32)
                for r in range(D // 16):
                    sl = pl.ds(r * 16, 16)
                    ff_v[b][j, sl] = ff_v[b][j, sl] * wb
                    ent_v[b][j, sl] = ent_v[b][j, sl] * wb
                return 0
            lax.fori_loop(0, K, _fact, 0)

            wait_out_lin(g - 1, 1 - b)

            @pl.when(g + 1 < NCHUNK)
            def _():
                for c in in_copies(g + 1, 1 - b):
                    c.wait()
                pltpu.async_copy(ent.at[idx_v[1 - b]], ent_v[1 - b],
                                 gsem[1 - b])

            pltpu.async_copy(ent_v[b],
                             out.at[pl.ds(gather_base + base, K)], osem[b])
            pltpu.async_copy(ff_v[b], accum.at[idx_v[b]], asem[b], add=True)
            if do_rel:
                pltpu.async_copy(ff_v[b], rel_accum.at[ridx_v[b]],
                                 asem[b], add=True)

        issue_in(0, 0)
        for c in in_copies(0, 0):
            c.wait()
        pltpu.async_copy(ent.at[idx_v[0]], ent_v[0], gsem[0])

        def outer(i, _):
            chunk_body(2 * i, 0)
            chunk_body(2 * i + 1, 1)
            return 0
        lax.fori_loop(0, NCHUNK // 2, outer, 0)
        wait_out_lin(NCHUNK - 1, 1)
        wait_adds(NCHUNK - 1, 1)

    @pl.when(cid == 0)
    def _():
        run(tails, NF, False)

    @pl.when(cid == 1)
    def _():
        run(heads, 0, True)

    plsc.subcore_barrier()

    ent_base = 2 * NF + cid * BME
    for k in range(-(-NEC // NS)):
        i = k * NS + sid

        @pl.when(i < NEC)
        def _():
            pltpu.sync_copy(accum.at[pl.ds(i * AC, AC)],
                            out.at[pl.ds(ent_base + i * AC, AC)])

    @pl.when(cid == 1)
    def _():
        for k in range(-(-NRC // NS)):
            i = k * NS + sid

            @pl.when(i < NRC)
            def _():
                pltpu.sync_copy(rel_accum.at[pl.ds(i * AC, AC)],
                                out.at[pl.ds(2 * NF + 2 * BME + i * AC, AC)])


def kernel(batch_heads, batch_rels, batch_tails, batch_ids, fact_ids,
           weight_list, entity_feat, fact_feat):
    del fact_ids
    i32 = jnp.int32
    return _gnn_sc(batch_heads.astype(i32), batch_rels.astype(i32),
                   batch_tails.astype(i32), batch_ids.astype(i32),
                   weight_list, entity_feat, fact_feat)
